# baseline (speedup 1.0000x reference)
import functools

import jax
import jax.numpy as jnp
from jax import lax
from jax.experimental import pallas as pl
from jax.experimental.pallas import tpu as pltpu
from jax.experimental.pallas import tpu_sc as plsc

VOCAB = 100000
DIM = 128
B = 1024
L = 50

NC = 2
NS = 16
NW = NC * NS
RPW = B // NW
NLANE = DIM // 16


CROWS = 2
CL = CROWS * L
NCHUNK = RPW // CROWS


def _sc_pool_body(idx_hbm, table_hbm, out_hbm, idx_v, buf0, buf1, out_v,
                  sem0, sem1):
    wid = lax.axis_index("s") * NC + lax.axis_index("c")
    pltpu.sync_copy(idx_hbm.at[wid], idx_v)

    def gather(c, buf, sem):
        return pltpu.make_async_copy(table_hbm.at[idx_v.at[c]], buf, sem)

    def accum(buf, r):
        def lane_acc(l, accs):
            a = tuple(accs[d] + buf[l, pl.ds(d * 16, 16)]
                      for d in range(NLANE))
            b = tuple(accs[NLANE + d] + buf[L + l, pl.ds(d * 16, 16)]
                      for d in range(NLANE))
            return a + b

        accs = tuple(buf[0, pl.ds(d * 16, 16)] for d in range(NLANE)) + \
               tuple(buf[L, pl.ds(d * 16, 16)] for d in range(NLANE))
        accs = lax.fori_loop(1, L, lane_acc, accs)
        for d in range(NLANE):
            out_v[r, pl.ds(d * 16, 16)] = accs[d] * (1.0 / L)
            out_v[r + 1, pl.ds(d * 16, 16)] = accs[NLANE + d] * (1.0 / L)

    gather(0, buf0, sem0).start()

    def step(k, carry):
        c0 = 2 * k
        gather(c0 + 1, buf1, sem1).start()
        gather(c0, buf0, sem0).wait()
        accum(buf0, 2 * c0)

        @pl.when(c0 + 2 < NCHUNK)
        def _():
            gather(c0 + 2, buf0, sem0).start()

        gather(c0 + 1, buf1, sem1).wait()
        accum(buf1, 2 * c0 + 2)
        return carry

    lax.fori_loop(0, NCHUNK // 2, step, 0)
    pltpu.sync_copy(out_v, out_hbm.at[pl.ds(wid * RPW, RPW)])


_sc_pool = functools.partial(
    pl.kernel,
    out_type=jax.ShapeDtypeStruct((B, DIM), jnp.float32),
    mesh=plsc.VectorSubcoreMesh(core_axis_name="c", subcore_axis_name="s"),
    scratch_types=[
        pltpu.VMEM((NCHUNK, CL), jnp.int32),
        pltpu.VMEM((CL, DIM), jnp.float32),
        pltpu.VMEM((CL, DIM), jnp.float32),
        pltpu.VMEM((RPW, DIM), jnp.float32),
        pltpu.SemaphoreType.DMA,
        pltpu.SemaphoreType.DMA,
    ],
)(_sc_pool_body)


VBLK = 1024
NBLK = pl.cdiv(VOCAB, VBLK)


def _mm_body(w_ref, pt_ref, b_ref, o_ref):
    acc = lax.dot_general(w_ref[...], pt_ref[...],
                          (((1,), (0,)), ((), ())),
                          preferred_element_type=jnp.float32)
    o_ref[...] = acc + b_ref[...]


def _decoder(pooled_t, W, b2d):
    out_t = pl.pallas_call(
        _mm_body,
        grid=(NBLK,),
        in_specs=[
            pl.BlockSpec((VBLK, DIM), lambda i: (i, 0)),
            pl.BlockSpec((DIM, B), lambda i: (0, 0)),
            pl.BlockSpec((VBLK, 1), lambda i: (i, 0)),
        ],
        out_specs=pl.BlockSpec((VBLK, B), lambda i: (i, 0)),
        out_shape=jax.ShapeDtypeStruct((VOCAB, B), jnp.float32),
    )(W, pooled_t, b2d)
    return out_t.T


def kernel(contextTsr, emb_table, W, b):
    idx = contextTsr.reshape(NW, NCHUNK, CL)
    pooled = _sc_pool(idx, emb_table)
    return _decoder(pooled.T, W, b.reshape(VOCAB, 1))


# speedup vs baseline: 1.0015x; 1.0015x over previous
import functools

import jax
import jax.numpy as jnp
from jax import lax
from jax.experimental import pallas as pl
from jax.experimental.pallas import tpu as pltpu
from jax.experimental.pallas import tpu_sc as plsc

VOCAB = 100000
DIM = 128
B = 1024
L = 50

NC = 2
NS = 16
NW = NC * NS
RPW = B // NW
NLANE = DIM // 16


CROWS = 2
CL = CROWS * L
NCHUNK = RPW // CROWS


def _sc_pool_body(idx_hbm, table_hbm, out_hbm, idx_v, buf0, buf1, out_v,
                  sem0, sem1):
    wid = lax.axis_index("s") * NC + lax.axis_index("c")
    pltpu.sync_copy(idx_hbm.at[wid], idx_v)

    def gather(c, buf, sem):
        return pltpu.make_async_copy(table_hbm.at[idx_v.at[c]], buf, sem)

    def accum(buf, r):
        def lane_acc(l, accs):
            a = tuple(accs[d] + buf[l, pl.ds(d * 16, 16)]
                      for d in range(NLANE))
            b = tuple(accs[NLANE + d] + buf[L + l, pl.ds(d * 16, 16)]
                      for d in range(NLANE))
            return a + b

        accs = tuple(buf[0, pl.ds(d * 16, 16)] for d in range(NLANE)) + \
               tuple(buf[L, pl.ds(d * 16, 16)] for d in range(NLANE))
        accs = lax.fori_loop(1, L, lane_acc, accs)
        for d in range(NLANE):
            out_v[r, pl.ds(d * 16, 16)] = accs[d] * (1.0 / L)
            out_v[r + 1, pl.ds(d * 16, 16)] = accs[NLANE + d] * (1.0 / L)

    gather(0, buf0, sem0).start()

    def step(k, carry):
        c0 = 2 * k
        gather(c0 + 1, buf1, sem1).start()
        gather(c0, buf0, sem0).wait()
        accum(buf0, 2 * c0)

        @pl.when(c0 + 2 < NCHUNK)
        def _():
            gather(c0 + 2, buf0, sem0).start()

        gather(c0 + 1, buf1, sem1).wait()
        accum(buf1, 2 * c0 + 2)
        return carry

    lax.fori_loop(0, NCHUNK // 2, step, 0)
    pltpu.sync_copy(out_v, out_hbm.at[pl.ds(wid * RPW, RPW)])


_sc_pool = functools.partial(
    pl.kernel,
    out_type=jax.ShapeDtypeStruct((B, DIM), jnp.float32),
    mesh=plsc.VectorSubcoreMesh(core_axis_name="c", subcore_axis_name="s"),
    scratch_types=[
        pltpu.VMEM((NCHUNK, CL), jnp.int32),
        pltpu.VMEM((CL, DIM), jnp.float32),
        pltpu.VMEM((CL, DIM), jnp.float32),
        pltpu.VMEM((RPW, DIM), jnp.float32),
        pltpu.SemaphoreType.DMA,
        pltpu.SemaphoreType.DMA,
    ],
)(_sc_pool_body)


VBLK = 1024
NBLK = pl.cdiv(VOCAB, VBLK)


def _mm_body(w_ref, pt_ref, b_ref, o_ref):
    acc = lax.dot_general(w_ref[...], pt_ref[...],
                          (((1,), (0,)), ((), ())),
                          preferred_element_type=jnp.float32)
    o_ref[...] = acc + b_ref[...]


def _decoder(pooled_t, W, b2d):
    out_t = pl.pallas_call(
        _mm_body,
        grid=(NBLK,),
        in_specs=[
            pl.BlockSpec((VBLK, DIM), lambda i: (i, 0)),
            pl.BlockSpec(memory_space=pltpu.VMEM),
            pl.BlockSpec((VBLK, 1), lambda i: (i, 0)),
        ],
        out_specs=pl.BlockSpec((VBLK, B), lambda i: (i, 0)),
        out_shape=jax.ShapeDtypeStruct((VOCAB, B), jnp.float32),
    )(W, pooled_t, b2d)
    return out_t.T


def kernel(contextTsr, emb_table, W, b):
    idx = contextTsr.reshape(NW, NCHUNK, CL)
    pooled = _sc_pool(idx, emb_table)
    return _decoder(pooled.T, W, b.reshape(VOCAB, 1))


# speedup vs baseline: 1.0785x; 1.0769x over previous
import functools

import jax
import jax.numpy as jnp
from jax import lax
from jax.experimental import pallas as pl
from jax.experimental.pallas import tpu as pltpu
from jax.experimental.pallas import tpu_sc as plsc

VOCAB = 100000
DIM = 128
B = 1024
L = 50

NC = 2
NS = 16
NW = NC * NS
RPW = B // NW
NLANE = DIM // 16


CROWS = 2
CL = CROWS * L
NCHUNK = RPW // CROWS


def _sc_pool_body(idx_hbm, table_hbm, out_hbm, idx_v, buf0, buf1, out_v,
                  sem0, sem1):
    wid = lax.axis_index("s") * NC + lax.axis_index("c")
    pltpu.sync_copy(idx_hbm.at[wid], idx_v)

    def gather(c, buf, sem):
        return pltpu.make_async_copy(table_hbm.at[idx_v.at[c]], buf, sem)

    def accum(buf, r):
        def lane_acc(l, accs):
            a = tuple(accs[d] + buf[l, pl.ds(d * 16, 16)]
                      for d in range(NLANE))
            b = tuple(accs[NLANE + d] + buf[L + l, pl.ds(d * 16, 16)]
                      for d in range(NLANE))
            return a + b

        accs = tuple(buf[0, pl.ds(d * 16, 16)] for d in range(NLANE)) + \
               tuple(buf[L, pl.ds(d * 16, 16)] for d in range(NLANE))
        accs = lax.fori_loop(1, L, lane_acc, accs)
        for d in range(NLANE):
            out_v[r, pl.ds(d * 16, 16)] = accs[d] * (1.0 / L)
            out_v[r + 1, pl.ds(d * 16, 16)] = accs[NLANE + d] * (1.0 / L)

    gather(0, buf0, sem0).start()

    def step(k, carry):
        c0 = 2 * k
        gather(c0 + 1, buf1, sem1).start()
        gather(c0, buf0, sem0).wait()
        accum(buf0, 2 * c0)

        @pl.when(c0 + 2 < NCHUNK)
        def _():
            gather(c0 + 2, buf0, sem0).start()

        gather(c0 + 1, buf1, sem1).wait()
        accum(buf1, 2 * c0 + 2)
        return carry

    lax.fori_loop(0, NCHUNK // 2, step, 0)
    pltpu.sync_copy(out_v, out_hbm.at[pl.ds(wid * RPW, RPW)])


_sc_pool = functools.partial(
    pl.kernel,
    out_type=jax.ShapeDtypeStruct((B, DIM), jnp.float32),
    mesh=plsc.VectorSubcoreMesh(core_axis_name="c", subcore_axis_name="s"),
    scratch_types=[
        pltpu.VMEM((NCHUNK, CL), jnp.int32),
        pltpu.VMEM((CL, DIM), jnp.float32),
        pltpu.VMEM((CL, DIM), jnp.float32),
        pltpu.VMEM((RPW, DIM), jnp.float32),
        pltpu.SemaphoreType.DMA,
        pltpu.SemaphoreType.DMA,
    ],
)(_sc_pool_body)


VBLK = 2048
NBLK = pl.cdiv(VOCAB, VBLK)


def _mm_body(w_ref, pt_ref, b_ref, o_ref):
    acc = lax.dot_general(w_ref[...], pt_ref[...],
                          (((1,), (0,)), ((), ())),
                          preferred_element_type=jnp.float32)
    o_ref[...] = acc + b_ref[...]


def _decoder(pooled_t, W, b2d):
    out_t = pl.pallas_call(
        _mm_body,
        grid=(NBLK,),
        in_specs=[
            pl.BlockSpec((VBLK, DIM), lambda i: (i, 0)),
            pl.BlockSpec(memory_space=pltpu.VMEM),
            pl.BlockSpec((VBLK, 1), lambda i: (i, 0)),
        ],
        out_specs=pl.BlockSpec((VBLK, B), lambda i: (i, 0)),
        out_shape=jax.ShapeDtypeStruct((VOCAB, B), jnp.float32),
    )(W, pooled_t, b2d)
    return out_t.T


def kernel(contextTsr, emb_table, W, b):
    idx = contextTsr.reshape(NW, NCHUNK, CL)
    pooled = _sc_pool(idx, emb_table)
    return _decoder(pooled.T, W, b.reshape(VOCAB, 1))


# speedup vs baseline: 1.0986x; 1.0186x over previous
import functools

import jax
import jax.numpy as jnp
from jax import lax
from jax.experimental import pallas as pl
from jax.experimental.pallas import tpu as pltpu
from jax.experimental.pallas import tpu_sc as plsc

VOCAB = 100000
DIM = 128
B = 1024
L = 50

NC = 2
NS = 16
NW = NC * NS
RPW = B // NW
NLANE = DIM // 16


CROWS = 2
CL = CROWS * L
NCHUNK = RPW // CROWS


def _sc_pool_body(idx_hbm, table_hbm, out_hbm, idx_v, buf0, buf1, out_v,
                  sem0, sem1):
    wid = lax.axis_index("s") * NC + lax.axis_index("c")
    pltpu.sync_copy(idx_hbm.at[wid], idx_v)

    def gather(c, buf, sem):
        return pltpu.make_async_copy(table_hbm.at[idx_v.at[c]], buf, sem)

    def accum(buf, r):
        def lane_acc(l, accs):
            a = tuple(accs[d] + buf[l, pl.ds(d * 16, 16)]
                      for d in range(NLANE))
            b = tuple(accs[NLANE + d] + buf[L + l, pl.ds(d * 16, 16)]
                      for d in range(NLANE))
            return a + b

        accs = tuple(buf[0, pl.ds(d * 16, 16)] for d in range(NLANE)) + \
               tuple(buf[L, pl.ds(d * 16, 16)] for d in range(NLANE))
        accs = lax.fori_loop(1, L, lane_acc, accs)
        for d in range(NLANE):
            out_v[r, pl.ds(d * 16, 16)] = accs[d] * (1.0 / L)
            out_v[r + 1, pl.ds(d * 16, 16)] = accs[NLANE + d] * (1.0 / L)

    gather(0, buf0, sem0).start()

    def step(k, carry):
        c0 = 2 * k
        gather(c0 + 1, buf1, sem1).start()
        gather(c0, buf0, sem0).wait()
        accum(buf0, 2 * c0)

        @pl.when(c0 + 2 < NCHUNK)
        def _():
            gather(c0 + 2, buf0, sem0).start()

        gather(c0 + 1, buf1, sem1).wait()
        accum(buf1, 2 * c0 + 2)
        return carry

    lax.fori_loop(0, NCHUNK // 2, step, 0)
    pltpu.sync_copy(out_v, out_hbm.at[pl.ds(wid * RPW, RPW)])


_sc_pool = functools.partial(
    pl.kernel,
    out_type=jax.ShapeDtypeStruct((B, DIM), jnp.float32),
    mesh=plsc.VectorSubcoreMesh(core_axis_name="c", subcore_axis_name="s"),
    scratch_types=[
        pltpu.VMEM((NCHUNK, CL), jnp.int32),
        pltpu.VMEM((CL, DIM), jnp.float32),
        pltpu.VMEM((CL, DIM), jnp.float32),
        pltpu.VMEM((RPW, DIM), jnp.float32),
        pltpu.SemaphoreType.DMA,
        pltpu.SemaphoreType.DMA,
    ],
)(_sc_pool_body)


VBLK = 4096
NBLK = pl.cdiv(VOCAB, VBLK)


def _mm_body(w_ref, pt_ref, b_ref, o_ref):
    acc = lax.dot_general(w_ref[...], pt_ref[...],
                          (((1,), (0,)), ((), ())),
                          preferred_element_type=jnp.float32)
    o_ref[...] = acc + b_ref[...]


def _decoder(pooled_t, W, b2d):
    out_t = pl.pallas_call(
        _mm_body,
        grid=(NBLK,),
        in_specs=[
            pl.BlockSpec((VBLK, DIM), lambda i: (i, 0)),
            pl.BlockSpec(memory_space=pltpu.VMEM),
            pl.BlockSpec((VBLK, 1), lambda i: (i, 0)),
        ],
        out_specs=pl.BlockSpec((VBLK, B), lambda i: (i, 0)),
        out_shape=jax.ShapeDtypeStruct((VOCAB, B), jnp.float32),
    )(W, pooled_t, b2d)
    return out_t.T


def kernel(contextTsr, emb_table, W, b):
    idx = contextTsr.reshape(NW, NCHUNK, CL)
    pooled = _sc_pool(idx, emb_table)
    return _decoder(pooled.T, W, b.reshape(VOCAB, 1))
